# Initial kernel scaffold; baseline (speedup 1.0000x reference)
import jax
import jax.numpy as jnp
from jax.experimental import pallas as pl


def kernel(data, segment_ids):
    raise NotImplementedError("write your pallas kernel here")


# speedup vs baseline: 4.5251x; 4.5251x over previous
import functools

import jax
import jax.numpy as jnp
from jax import lax
from jax.experimental import pallas as pl
from jax.experimental.pallas import tpu as pltpu
from jax.experimental.pallas import tpu_sc as plsc

N_ROWS = 320000
D_FEAT = 128
N_SEG = 10000
SEG_PAD = 10240

NC = 2
NS = 16
NW = NC * NS
ROWS_PER_TILE = N_ROWS // NW
CHUNK = 80
NCHUNK = ROWS_PER_TILE // CHUNK
SEG_PER_TILE = SEG_PAD // NS

_mesh = plsc.VectorSubcoreMesh(core_axis_name="c", subcore_axis_name="s")


@functools.partial(
    pl.kernel,
    mesh=_mesh,
    out_type=jax.ShapeDtypeStruct((NC, SEG_PAD, D_FEAT), jnp.float32),
    scratch_types=[
        pltpu.VMEM((NCHUNK, CHUNK), jnp.int32),
        pltpu.VMEM((CHUNK, D_FEAT), jnp.float32),
        pltpu.VMEM_SHARED((SEG_PAD, D_FEAT), jnp.float32),
    ],
)
def _segment_sum_sc(data_hbm, seg_hbm, zeros_hbm, out_hbm, ids_v, buf, acc):
    c = lax.axis_index("c")
    s = lax.axis_index("s")
    wid = c * NS + s

    pltpu.sync_copy(zeros_hbm, acc.at[pl.ds(s * SEG_PER_TILE, SEG_PER_TILE)])
    plsc.subcore_barrier()

    pltpu.sync_copy(seg_hbm.at[wid], ids_v)

    def body(j, carry):
        base = wid * ROWS_PER_TILE + j * CHUNK
        pltpu.sync_copy(data_hbm.at[pl.ds(base, CHUNK)], buf)
        pltpu.sync_copy(buf, acc.at[ids_v.at[j]], add=True)
        return carry

    lax.fori_loop(0, NCHUNK, body, 0)
    plsc.subcore_barrier()

    pltpu.sync_copy(
        acc.at[pl.ds(s * SEG_PER_TILE, SEG_PER_TILE)],
        out_hbm.at[c, pl.ds(s * SEG_PER_TILE, SEG_PER_TILE)],
    )


def _combine_body(p_ref, o_ref):
    o_ref[...] = p_ref[0] + p_ref[1]


def _combine(partials):
    nblk = 10
    rows = N_SEG // nblk
    return pl.pallas_call(
        _combine_body,
        out_shape=jax.ShapeDtypeStruct((N_SEG, D_FEAT), jnp.float32),
        grid=(nblk,),
        in_specs=[pl.BlockSpec((NC, rows, D_FEAT), lambda i: (0, i, 0))],
        out_specs=pl.BlockSpec((rows, D_FEAT), lambda i: (i, 0)),
    )(partials)


def kernel(data, segment_ids):
    seg3d = segment_ids.astype(jnp.int32).reshape(NW, NCHUNK, CHUNK)
    zeros = jnp.zeros((SEG_PER_TILE, D_FEAT), jnp.float32)
    partials = _segment_sum_sc(data, seg3d, zeros)
    return _combine(partials)


# speedup vs baseline: 7.0972x; 1.5684x over previous
import functools

import jax
import jax.numpy as jnp
from jax import lax
from jax.experimental import pallas as pl
from jax.experimental.pallas import tpu as pltpu
from jax.experimental.pallas import tpu_sc as plsc

N_ROWS = 320000
D_FEAT = 128
N_SEG = 10000
SEG_PAD = 10240

NC = 2
NS = 16
NW = NC * NS
ROWS_PER_TILE = N_ROWS // NW
CHUNK = 80
NCHUNK = ROWS_PER_TILE // CHUNK
SEG_PER_TILE = SEG_PAD // NS

_mesh = plsc.VectorSubcoreMesh(core_axis_name="c", subcore_axis_name="s")


@functools.partial(
    pl.kernel,
    mesh=_mesh,
    out_type=jax.ShapeDtypeStruct((NC, SEG_PAD, D_FEAT), jnp.float32),
    scratch_types=[
        pltpu.VMEM((NCHUNK, CHUNK), jnp.int32),
        pltpu.VMEM((CHUNK, D_FEAT), jnp.float32),
        pltpu.VMEM((CHUNK, D_FEAT), jnp.float32),
        pltpu.VMEM_SHARED((SEG_PAD, D_FEAT), jnp.float32),
        pltpu.SemaphoreType.DMA,
        pltpu.SemaphoreType.DMA,
    ],
)
def _segment_sum_sc(data_hbm, seg_hbm, zeros_hbm, out_hbm, ids_v, buf0, buf1,
                    acc, sem0, sem1):
    c = lax.axis_index("c")
    s = lax.axis_index("s")
    wid = c * NS + s
    base0 = wid * ROWS_PER_TILE

    pltpu.sync_copy(zeros_hbm, acc.at[pl.ds(s * SEG_PER_TILE, SEG_PER_TILE)])
    plsc.subcore_barrier()

    pltpu.sync_copy(seg_hbm.at[wid], ids_v)

    pltpu.async_copy(data_hbm.at[pl.ds(base0, CHUNK)], buf0, sem0)

    def body(i, carry):
        ch0 = 2 * i
        pltpu.async_copy(
            data_hbm.at[pl.ds(base0 + (ch0 + 1) * CHUNK, CHUNK)], buf1, sem1)
        pltpu.make_async_copy(
            data_hbm.at[pl.ds(base0 + ch0 * CHUNK, CHUNK)], buf0, sem0).wait()
        pltpu.sync_copy(buf0, acc.at[ids_v.at[ch0]], add=True)
        pltpu.async_copy(
            data_hbm.at[pl.ds(base0 + (ch0 + 2) * CHUNK, CHUNK)], buf0, sem0)
        pltpu.make_async_copy(
            data_hbm.at[pl.ds(base0 + (ch0 + 1) * CHUNK, CHUNK)], buf1,
            sem1).wait()
        pltpu.sync_copy(buf1, acc.at[ids_v.at[ch0 + 1]], add=True)
        return carry

    lax.fori_loop(0, (NCHUNK - 1) // 2, body, 0)
    pltpu.make_async_copy(
        data_hbm.at[pl.ds(base0 + (NCHUNK - 1) * CHUNK, CHUNK)], buf0,
        sem0).wait()
    pltpu.sync_copy(buf0, acc.at[ids_v.at[NCHUNK - 1]], add=True)
    plsc.subcore_barrier()

    pltpu.sync_copy(
        acc.at[pl.ds(s * SEG_PER_TILE, SEG_PER_TILE)],
        out_hbm.at[c, pl.ds(s * SEG_PER_TILE, SEG_PER_TILE)],
    )


def _combine_body(p_ref, o_ref):
    o_ref[...] = p_ref[0] + p_ref[1]


def _combine(partials):
    nblk = 10
    rows = N_SEG // nblk
    return pl.pallas_call(
        _combine_body,
        out_shape=jax.ShapeDtypeStruct((N_SEG, D_FEAT), jnp.float32),
        grid=(nblk,),
        in_specs=[pl.BlockSpec((NC, rows, D_FEAT), lambda i: (0, i, 0))],
        out_specs=pl.BlockSpec((rows, D_FEAT), lambda i: (i, 0)),
    )(partials)


def kernel(data, segment_ids):
    seg3d = segment_ids.astype(jnp.int32).reshape(NW, NCHUNK, CHUNK)
    zeros = jnp.zeros((SEG_PER_TILE, D_FEAT), jnp.float32)
    partials = _segment_sum_sc(data, seg3d, zeros)
    return _combine(partials)
